# Initial kernel scaffold; baseline (speedup 1.0000x reference)
import functools

import jax
import jax.numpy as jnp
from jax import lax
from jax.experimental import pallas as pl
from jax.experimental.pallas import tpu as pltpu
from jax.experimental.pallas import tpu_sc as plsc

NUM_EMB = 8192
D = 64
B_ROWS = 4096
T = 200
B_FLAT = B_ROWS * T

NC = 2
NS = 16
NW = NC * NS
B_PER_W = B_FLAT // NW
CHUNK = 128
CHUNKS = B_PER_W // CHUNK

_mesh = plsc.VectorSubcoreMesh(core_axis_name="c", subcore_axis_name="s")


@functools.partial(
    pl.kernel,
    out_type=jax.ShapeDtypeStruct((B_FLAT, D), jnp.float32),
    mesh=_mesh,
    scratch_types=[
        pltpu.VMEM((CHUNKS, CHUNK), jnp.int32),
        pltpu.VMEM((CHUNK, D), jnp.float32),
        pltpu.SemaphoreType.DMA,
    ],
)
def _gather_rows(table_hbm, idx_hbm, out_hbm, idx_v, rows_v, sem):
    wid = lax.axis_index("s") * NC + lax.axis_index("c")
    base = wid * B_PER_W
    pltpu.sync_copy(idx_hbm.at[wid], idx_v)

    def step(j, carry):
        pltpu.async_copy(table_hbm.at[idx_v.at[j]], rows_v, sem).wait()
        pltpu.sync_copy(rows_v, out_hbm.at[pl.ds(base + j * CHUNK, CHUNK)])
        return carry

    lax.fori_loop(0, CHUNKS, step, 0)


def kernel(idx, weight):
    idx3 = idx.reshape(NW, CHUNKS, CHUNK).astype(jnp.int32)
    out = _gather_rows(weight, idx3)
    return out.reshape(B_ROWS, T, D)


# speedup vs baseline: 4.1696x; 4.1696x over previous
import functools

import jax
import jax.numpy as jnp
from jax import lax
from jax.experimental import pallas as pl
from jax.experimental.pallas import tpu as pltpu
from jax.experimental.pallas import tpu_sc as plsc

NUM_EMB = 8192
D = 64
B_ROWS = 4096
T = 200
B_FLAT = B_ROWS * T

NC = 2
NS = 16
NW = NC * NS
B_PER_W = B_FLAT // NW
CHUNK = 128
CHUNKS = B_PER_W // CHUNK

_mesh = plsc.VectorSubcoreMesh(core_axis_name="c", subcore_axis_name="s")


@functools.partial(
    pl.kernel,
    out_type=jax.ShapeDtypeStruct((B_FLAT, D), jnp.float32),
    mesh=_mesh,
    scratch_types=[
        pltpu.VMEM((CHUNKS, CHUNK), jnp.int32),
        pltpu.VMEM((CHUNK, D), jnp.float32),
        pltpu.SemaphoreType.DMA,
    ],
    compiler_params=pltpu.CompilerParams(use_tc_tiling_on_sc=False),
)
def _gather_rows(table_hbm, idx_hbm, out_hbm, idx_v, rows_v, sem):
    wid = lax.axis_index("s") * NC + lax.axis_index("c")
    base = wid * B_PER_W
    pltpu.sync_copy(idx_hbm.at[wid], idx_v)

    def step(j, carry):
        pltpu.async_copy(table_hbm.at[idx_v.at[j]], rows_v, sem).wait()
        pltpu.sync_copy(rows_v, out_hbm.at[pl.ds(base + j * CHUNK, CHUNK)])
        return carry

    lax.fori_loop(0, CHUNKS, step, 0)


def kernel(idx, weight):
    idx3 = idx.reshape(NW, CHUNKS, CHUNK).astype(jnp.int32)
    out = _gather_rows(weight, idx3)
    return out.reshape(B_ROWS, T, D)


# speedup vs baseline: 4.9206x; 1.1801x over previous
import functools

import jax
import jax.numpy as jnp
from jax import lax
from jax.experimental import pallas as pl
from jax.experimental.pallas import tpu as pltpu
from jax.experimental.pallas import tpu_sc as plsc

NUM_EMB = 8192
D = 64
B_ROWS = 4096
T = 200
B_FLAT = B_ROWS * T

NC = 2
NS = 16
NW = NC * NS
B_PER_W = B_FLAT // NW
CHUNK = 128
CHUNKS = B_PER_W // CHUNK

K = 4
GROUP = K * CHUNK
S = CHUNKS // K

_mesh = plsc.VectorSubcoreMesh(core_axis_name="c", subcore_axis_name="s")


@functools.partial(
    pl.kernel,
    out_type=jax.ShapeDtypeStruct((B_FLAT, D), jnp.float32),
    mesh=_mesh,
    scratch_types=[
        pltpu.VMEM((CHUNKS, CHUNK), jnp.int32),
        pltpu.VMEM((GROUP, D), jnp.float32),
        pltpu.VMEM((GROUP, D), jnp.float32),
        pltpu.SemaphoreType.DMA,
        pltpu.SemaphoreType.DMA,
        pltpu.SemaphoreType.DMA,
        pltpu.SemaphoreType.DMA,
    ],
    compiler_params=pltpu.CompilerParams(use_tc_tiling_on_sc=False),
)
def _gather_rows(table_hbm, idx_hbm, out_hbm, idx_v, rows_a, rows_b,
                 gsa, gsb, osa, osb):
    wid = lax.axis_index("s") * NC + lax.axis_index("c")
    base = wid * B_PER_W
    pltpu.sync_copy(idx_hbm.at[wid], idx_v)

    def fire_gather(rows, gsem, s):
        for b in range(K):
            pltpu.async_copy(table_hbm.at[idx_v.at[s * K + b]],
                             rows.at[pl.ds(b * CHUNK, CHUNK)], gsem)

    def drain_gather(rows, gsem):
        for b in range(K):
            pltpu.make_async_copy(table_hbm.at[idx_v.at[0]],
                                  rows.at[pl.ds(b * CHUNK, CHUNK)], gsem).wait()

    def fire_out(rows, osem, s):
        pltpu.async_copy(rows, out_hbm.at[pl.ds(base + s * GROUP, GROUP)], osem)

    def drain_out(rows, osem):
        pltpu.make_async_copy(rows, out_hbm.at[pl.ds(base, GROUP)], osem).wait()

    fire_gather(rows_a, gsa, 0)

    def super_step(p, carry):
        s0 = 2 * p
        drain_gather(rows_a, gsa)
        fire_out(rows_a, osa, s0)

        @pl.when(p > 0)
        def _():
            drain_out(rows_b, osb)

        fire_gather(rows_b, gsb, s0 + 1)
        drain_gather(rows_b, gsb)
        fire_out(rows_b, osb, s0 + 1)
        drain_out(rows_a, osa)

        @pl.when(s0 + 2 < S)
        def _():
            fire_gather(rows_a, gsa, s0 + 2)

        return carry

    lax.fori_loop(0, S // 2, super_step, 0)
    drain_out(rows_b, osb)


def kernel(idx, weight):
    idx3 = idx.reshape(NW, CHUNKS, CHUNK).astype(jnp.int32)
    out = _gather_rows(weight, idx3)
    return out.reshape(B_ROWS, T, D)


# speedup vs baseline: 6.0411x; 1.2277x over previous
import functools

import jax
import jax.numpy as jnp
from jax import lax
from jax.experimental import pallas as pl
from jax.experimental.pallas import tpu as pltpu
from jax.experimental.pallas import tpu_sc as plsc

NUM_EMB = 8192
D = 64
B_ROWS = 4096
T = 200
L = 16

NC = 2
NS = 16
NW = NC * NS

DH = 8
TQ = NW // DH
T_PER_W = T // TQ
BH = 32
HB = BH // 2

_mesh = plsc.VectorSubcoreMesh(core_axis_name="c", subcore_axis_name="s")


@functools.partial(
    pl.kernel,
    out_type=jax.ShapeDtypeStruct((T, DH, BH, 8, 128), jnp.float32),
    mesh=_mesh,
    scratch_types=[
        pltpu.VMEM((8, NUM_EMB), jnp.float32),
        pltpu.VMEM((2, B_ROWS), jnp.int32),
        pltpu.VMEM((HB, 8, 128), jnp.float32),
        pltpu.VMEM((HB, 8, 128), jnp.float32),
        pltpu.SemaphoreType.DMA,
        pltpu.SemaphoreType.DMA,
        pltpu.SemaphoreType.DMA,
    ],
    compiler_params=pltpu.CompilerParams(needs_layout_passes=False),
)
def _gather_t(tableT_hbm, idxT_hbm, out_hbm, slab_v, idx_v, ob_a, ob_b,
              isem, osa, osb):
    wid = lax.axis_index("s") * NC + lax.axis_index("c")
    dh = wid // TQ
    t0 = (wid % TQ) * T_PER_W

    pltpu.sync_copy(tableT_hbm.at[pl.ds(dh * 8, 8)], slab_v)
    pltpu.async_copy(idxT_hbm.at[t0], idx_v.at[0], isem)

    dl_splats = [jnp.full((L,), dl, jnp.int32) for dl in range(8)]

    def fill_tile(tb, bh, ob, bh_loc):
        for g in range(128 // L):
            iv = idx_v[tb, pl.ds(bh * 128 + g * L, L)]
            for dl in range(8):
                ob[bh_loc, dl, pl.ds(g * L, L)] = plsc.load_gather(
                    slab_v, [dl_splats[dl], iv])

    def t_step(ti, carry):
        t = t0 + ti
        tb = lax.rem(ti, 2)
        pltpu.make_async_copy(idxT_hbm.at[t0], idx_v.at[0], isem).wait()

        @pl.when(ti + 1 < T_PER_W)
        def _():
            pltpu.async_copy(idxT_hbm.at[t + 1], idx_v.at[1 - tb], isem)

        def half(ob, osem, h):
            @pl.when(ti > 0)
            def _():
                pltpu.make_async_copy(
                    ob, out_hbm.at[t, dh, pl.ds(h * HB, HB)], osem).wait()

            def tile_body(bh_loc, c):
                fill_tile(tb, h * HB + bh_loc, ob, bh_loc)
                return c

            lax.fori_loop(0, HB, tile_body, 0)
            pltpu.async_copy(ob, out_hbm.at[t, dh, pl.ds(h * HB, HB)], osem)

        half(ob_a, osa, 0)
        half(ob_b, osb, 1)
        return carry

    lax.fori_loop(0, T_PER_W, t_step, 0)
    pltpu.make_async_copy(ob_a, out_hbm.at[t0, dh, pl.ds(0, HB)], osa).wait()
    pltpu.make_async_copy(ob_b, out_hbm.at[t0, dh, pl.ds(HB, HB)], osb).wait()


def kernel(idx, weight):
    tableT = weight.T
    idxT = idx.T.astype(jnp.int32)
    out5 = _gather_t(tableT, idxT)
    return out5.transpose(2, 4, 0, 1, 3).reshape(B_ROWS, T, D)


# speedup vs baseline: 12.6420x; 2.0927x over previous
import functools

import jax
import jax.numpy as jnp
from jax import lax
from jax.experimental import pallas as pl
from jax.experimental.pallas import tpu as pltpu
from jax.experimental.pallas import tpu_sc as plsc

NUM_EMB = 8192
D = 64
B_ROWS = 4096
T = 200
L = 16

NC = 2
NS = 16
NW = NC * NS

DH = 8
TQ = NW // DH
T_PER_W = T // TQ
BH = 32
HB = BH // 2

_mesh = plsc.VectorSubcoreMesh(core_axis_name="c", subcore_axis_name="s")


@functools.partial(
    pl.kernel,
    out_type=jax.ShapeDtypeStruct((T, DH, BH, 8, 128), jnp.float32),
    mesh=_mesh,
    scratch_types=[
        pltpu.VMEM((8, NUM_EMB), jnp.float32),
        pltpu.VMEM((2, B_ROWS), jnp.int32),
        pltpu.VMEM((HB, 8, 128), jnp.float32),
        pltpu.VMEM((HB, 8, 128), jnp.float32),
        pltpu.SemaphoreType.DMA,
        pltpu.SemaphoreType.DMA,
        pltpu.SemaphoreType.DMA,
    ],
    compiler_params=pltpu.CompilerParams(needs_layout_passes=False),
)
def _gather_t(tableT_hbm, idxT_hbm, out_hbm, slab_v, idx_v, ob_a, ob_b,
              isem, osa, osb):
    wid = lax.axis_index("s") * NC + lax.axis_index("c")
    dh = wid // TQ
    t0 = (wid % TQ) * T_PER_W

    pltpu.sync_copy(tableT_hbm.at[pl.ds(dh * 8, 8)], slab_v)
    pltpu.async_copy(idxT_hbm.at[t0], idx_v.at[0], isem)

    dl_splats = [jnp.full((L,), dl, jnp.int32) for dl in range(8)]

    def fill_tile(tb, bh, ob, bh_loc):
        vals_prev = None
        for g in range(128 // L):
            iv = idx_v[tb, pl.ds(bh * 128 + g * L, L)]
            vals = [plsc.load_gather(slab_v, [dl_splats[dl], iv])
                    for dl in range(8)]
            if vals_prev is not None:
                for dl in range(8):
                    ob[bh_loc, dl, pl.ds((g - 1) * L, L)] = vals_prev[dl]
            vals_prev = vals
        for dl in range(8):
            ob[bh_loc, dl, pl.ds((128 // L - 1) * L, L)] = vals_prev[dl]

    def t_step(ti, carry):
        t = t0 + ti
        tb = lax.rem(ti, 2)
        pltpu.make_async_copy(idxT_hbm.at[t0], idx_v.at[0], isem).wait()

        @pl.when(ti + 1 < T_PER_W)
        def _():
            pltpu.async_copy(idxT_hbm.at[t + 1], idx_v.at[1 - tb], isem)

        def half(ob, osem, h):
            @pl.when(ti > 0)
            def _():
                pltpu.make_async_copy(
                    ob, out_hbm.at[t, dh, pl.ds(h * HB, HB)], osem).wait()

            def tile_body(bh_loc, c):
                fill_tile(tb, h * HB + bh_loc, ob, bh_loc)
                return c

            lax.fori_loop(0, HB, tile_body, 0)
            pltpu.async_copy(ob, out_hbm.at[t, dh, pl.ds(h * HB, HB)], osem)

        half(ob_a, osa, 0)
        half(ob_b, osb, 1)
        return carry

    lax.fori_loop(0, T_PER_W, t_step, 0)
    pltpu.make_async_copy(ob_a, out_hbm.at[t0, dh, pl.ds(0, HB)], osa).wait()
    pltpu.make_async_copy(ob_b, out_hbm.at[t0, dh, pl.ds(HB, HB)], osb).wait()


def kernel(idx, weight):
    tableT = weight.T
    idxT = idx.T.astype(jnp.int32)
    out5 = _gather_t(tableT, idxT)
    return out5.transpose(2, 4, 0, 1, 3).reshape(B_ROWS, T, D)
